# baseline (speedup 1.0000x reference)
import functools

import jax
import jax.numpy as jnp
from jax import lax
from jax.experimental import pallas as pl
from jax.experimental.pallas import tpu as pltpu
from jax.experimental.pallas import tpu_sc as plsc

N_NODES = 10000
N_EDGES = 320000
D_IN = 128
D_HID = 256
D_OUT = 64

NC = 2
NS = 16
NW = NC * NS
EPW = N_EDGES // NW
CHUNK = 80
NCHUNK = EPW // CHUNK
HALF = 64
PAD_NODES = 10240
ROWS_PT = PAD_NODES // NS

_sc_mesh = plsc.VectorSubcoreMesh(
    core_axis_name="c", subcore_axis_name="s", num_cores=NC, num_subcores=NS
)


@functools.partial(
    pl.kernel,
    out_type=jax.ShapeDtypeStruct((NC, PAD_NODES, D_IN), jnp.float32),
    mesh=_sc_mesh,
    scratch_types=[
        pltpu.VMEM((HALF, CHUNK), jnp.int32),
        pltpu.VMEM((HALF, CHUNK), jnp.int32),
        pltpu.VMEM((2, CHUNK, D_IN), jnp.float32),
        pltpu.VMEM_SHARED((PAD_NODES, D_IN), jnp.float32),
        pltpu.SemaphoreType.DMA,
        pltpu.SemaphoreType.DMA,
    ],
)
def _gin_aggregate(x_hbm, src_hbm, dst_hbm, zeros_hbm, out_hbm,
                   sidx, didx, rows, acc, gsem, ssem):
    c = lax.axis_index("c")
    s = lax.axis_index("s")
    wid = s * NC + c

    pltpu.sync_copy(zeros_hbm, acc.at[pl.ds(s * ROWS_PT, ROWS_PT)])

    def fire_gather(j, b):
        pltpu.async_copy(x_hbm.at[sidx.at[j]], rows.at[b], gsem)

    def fire_scatter(j, b):
        pltpu.async_copy(rows.at[b], acc.at[didx.at[j]], ssem, add=True)

    def drain(sem):
        pltpu.make_async_copy(rows.at[0], acc.at[didx.at[0]], sem).wait()

    def run_span(base, count):
        pltpu.sync_copy(src_hbm.at[wid, pl.ds(base, count)],
                        sidx.at[pl.ds(0, count)])
        pltpu.sync_copy(dst_hbm.at[wid, pl.ds(base, count)],
                        didx.at[pl.ds(0, count)])

        fire_gather(0, 0)
        drain(gsem)
        fire_scatter(0, 0)
        fire_gather(1, 1)

        def body(j, carry):
            g = lax.rem(j, 2)
            drain(gsem)
            drain(ssem)
            fire_scatter(j, g)
            fire_gather(j + 1, 1 - g)
            return carry

        lax.fori_loop(1, count - 1, body, 0)

        gl = (count - 1) % 2
        drain(gsem)
        drain(ssem)
        fire_scatter(count - 1, gl)
        drain(ssem)

    plsc.subcore_barrier()
    run_span(0, HALF)
    run_span(HALF, NCHUNK - HALF)
    plsc.subcore_barrier()

    pltpu.sync_copy(
        acc.at[pl.ds(s * ROWS_PT, ROWS_PT)],
        out_hbm.at[c, pl.ds(s * ROWS_PT, ROWS_PT)],
    )


ROW_BLK = 400


def _mlp_body(x_ref, p_ref, w1_ref, b1_ref, w2_ref, b2_ref, o_ref):
    h = x_ref[...] + p_ref[0] + p_ref[1]
    h1 = jnp.dot(h, w1_ref[...], preferred_element_type=jnp.float32)
    h1 = jnp.maximum(h1 + b1_ref[...], 0.0)
    h2 = jnp.dot(h1, w2_ref[...], preferred_element_type=jnp.float32)
    h2 = jnp.maximum(h2 + b2_ref[...], 0.0)
    m = jnp.max(h2, axis=1, keepdims=True)
    e = h2 - m
    lse = jnp.log(jnp.sum(jnp.exp(e), axis=1, keepdims=True))
    o_ref[...] = e - lse


def kernel(x, edge_index, W1, b1, W2, b2):
    src = edge_index[0].astype(jnp.int32).reshape(NW, NCHUNK, CHUNK)
    dst = edge_index[1].astype(jnp.int32).reshape(NW, NCHUNK, CHUNK)
    zeros = jnp.zeros((ROWS_PT, D_IN), jnp.float32)

    parts = _gin_aggregate(x, src, dst, zeros)

    grid = (N_NODES // ROW_BLK,)
    out = pl.pallas_call(
        _mlp_body,
        grid=grid,
        in_specs=[
            pl.BlockSpec((ROW_BLK, D_IN), lambda i: (i, 0)),
            pl.BlockSpec((NC, ROW_BLK, D_IN), lambda i: (0, i, 0)),
            pl.BlockSpec((D_IN, D_HID), lambda i: (0, 0)),
            pl.BlockSpec((1, D_HID), lambda i: (0, 0)),
            pl.BlockSpec((D_HID, D_OUT), lambda i: (0, 0)),
            pl.BlockSpec((1, D_OUT), lambda i: (0, 0)),
        ],
        out_specs=pl.BlockSpec((ROW_BLK, D_OUT), lambda i: (i, 0)),
        out_shape=jax.ShapeDtypeStruct((N_NODES, D_OUT), jnp.float32),
    )(x, parts, W1, b1.reshape(1, D_HID), W2, b2.reshape(1, D_OUT))
    return out


# speedup vs baseline: 1.1285x; 1.1285x over previous
import functools

import jax
import jax.numpy as jnp
from jax import lax
from jax.experimental import pallas as pl
from jax.experimental.pallas import tpu as pltpu
from jax.experimental.pallas import tpu_sc as plsc

N_NODES = 10000
N_EDGES = 320000
D_IN = 128
D_HID = 256
D_OUT = 64

NC = 2
NS = 16
NW = NC * NS
EPW = N_EDGES // NW
CHUNK = 125
NCHUNK = EPW // CHUNK
HALF = 40
PAD_NODES = 10240
ROWS_PT = PAD_NODES // NS

_sc_mesh = plsc.VectorSubcoreMesh(
    core_axis_name="c", subcore_axis_name="s", num_cores=NC, num_subcores=NS
)


@functools.partial(
    pl.kernel,
    out_type=jax.ShapeDtypeStruct((NC, PAD_NODES, D_IN), jnp.float32),
    mesh=_sc_mesh,
    scratch_types=[
        pltpu.VMEM((HALF, CHUNK), jnp.int32),
        pltpu.VMEM((HALF, CHUNK), jnp.int32),
        pltpu.VMEM((2, CHUNK, D_IN), jnp.float32),
        pltpu.VMEM_SHARED((PAD_NODES, D_IN), jnp.float32),
        pltpu.SemaphoreType.DMA,
        pltpu.SemaphoreType.DMA,
    ],
)
def _gin_aggregate(x_hbm, src_hbm, dst_hbm, zeros_hbm, out_hbm,
                   sidx, didx, rows, acc, gsem, ssem):
    c = lax.axis_index("c")
    s = lax.axis_index("s")
    wid = s * NC + c

    pltpu.sync_copy(zeros_hbm, acc.at[pl.ds(s * ROWS_PT, ROWS_PT)])

    def fire_gather(j, b):
        pltpu.async_copy(x_hbm.at[sidx.at[j]], rows.at[b], gsem)

    def fire_scatter(j, b):
        pltpu.async_copy(rows.at[b], acc.at[didx.at[j]], ssem, add=True)

    def drain(sem):
        pltpu.make_async_copy(rows.at[0], acc.at[didx.at[0]], sem).wait()

    def run_span(base, count):
        pltpu.sync_copy(src_hbm.at[wid, pl.ds(base, count)],
                        sidx.at[pl.ds(0, count)])
        pltpu.sync_copy(dst_hbm.at[wid, pl.ds(base, count)],
                        didx.at[pl.ds(0, count)])

        fire_gather(0, 0)
        drain(gsem)
        fire_scatter(0, 0)
        fire_gather(1, 1)

        def body(j, carry):
            g = lax.rem(j, 2)
            drain(gsem)
            drain(ssem)
            fire_scatter(j, g)
            fire_gather(j + 1, 1 - g)
            return carry

        lax.fori_loop(1, count - 1, body, 0)

        gl = (count - 1) % 2
        drain(gsem)
        drain(ssem)
        fire_scatter(count - 1, gl)
        drain(ssem)

    plsc.subcore_barrier()
    run_span(0, HALF)
    run_span(HALF, NCHUNK - HALF)
    plsc.subcore_barrier()

    pltpu.sync_copy(
        acc.at[pl.ds(s * ROWS_PT, ROWS_PT)],
        out_hbm.at[c, pl.ds(s * ROWS_PT, ROWS_PT)],
    )


ROW_BLK = 400


def _mlp_body(x_ref, p_ref, w1_ref, b1_ref, w2_ref, b2_ref, o_ref):
    h = x_ref[...] + p_ref[0] + p_ref[1]
    h1 = jnp.dot(h, w1_ref[...], preferred_element_type=jnp.float32)
    h1 = jnp.maximum(h1 + b1_ref[...], 0.0)
    h2 = jnp.dot(h1, w2_ref[...], preferred_element_type=jnp.float32)
    h2 = jnp.maximum(h2 + b2_ref[...], 0.0)
    m = jnp.max(h2, axis=1, keepdims=True)
    e = h2 - m
    lse = jnp.log(jnp.sum(jnp.exp(e), axis=1, keepdims=True))
    o_ref[...] = e - lse


def kernel(x, edge_index, W1, b1, W2, b2):
    src = edge_index[0].astype(jnp.int32).reshape(NW, NCHUNK, CHUNK)
    dst = edge_index[1].astype(jnp.int32).reshape(NW, NCHUNK, CHUNK)
    zeros = jnp.zeros((ROWS_PT, D_IN), jnp.float32)

    parts = _gin_aggregate(x, src, dst, zeros)

    grid = (N_NODES // ROW_BLK,)
    out = pl.pallas_call(
        _mlp_body,
        grid=grid,
        in_specs=[
            pl.BlockSpec((ROW_BLK, D_IN), lambda i: (i, 0)),
            pl.BlockSpec((NC, ROW_BLK, D_IN), lambda i: (0, i, 0)),
            pl.BlockSpec((D_IN, D_HID), lambda i: (0, 0)),
            pl.BlockSpec((1, D_HID), lambda i: (0, 0)),
            pl.BlockSpec((D_HID, D_OUT), lambda i: (0, 0)),
            pl.BlockSpec((1, D_OUT), lambda i: (0, 0)),
        ],
        out_specs=pl.BlockSpec((ROW_BLK, D_OUT), lambda i: (i, 0)),
        out_shape=jax.ShapeDtypeStruct((N_NODES, D_OUT), jnp.float32),
    )(x, parts, W1, b1.reshape(1, D_HID), W2, b2.reshape(1, D_OUT))
    return out


# speedup vs baseline: 1.1910x; 1.0554x over previous
import functools

import jax
import jax.numpy as jnp
from jax import lax
from jax.experimental import pallas as pl
from jax.experimental.pallas import tpu as pltpu
from jax.experimental.pallas import tpu_sc as plsc

N_NODES = 10000
N_EDGES = 320000
D_IN = 128
D_HID = 256
D_OUT = 64

NC = 2
NS = 16
NW = NC * NS
EPW = N_EDGES // NW
CHUNK = 125
NCHUNK = EPW // CHUNK
HALF = 40
PAD_NODES = 10240
ROWS_PT = PAD_NODES // NS

_sc_mesh = plsc.VectorSubcoreMesh(
    core_axis_name="c", subcore_axis_name="s", num_cores=NC, num_subcores=NS
)


@functools.partial(
    pl.kernel,
    out_type=jax.ShapeDtypeStruct((NC, PAD_NODES, D_IN), jnp.float32),
    mesh=_sc_mesh,
    scratch_types=[
        pltpu.VMEM((HALF, CHUNK), jnp.int32),
        pltpu.VMEM((HALF, CHUNK), jnp.int32),
        pltpu.VMEM((2, CHUNK, D_IN), jnp.float32),
        pltpu.VMEM_SHARED((PAD_NODES, D_IN), jnp.float32),
        pltpu.SemaphoreType.DMA,
        pltpu.SemaphoreType.DMA,
    ],
)
def _gin_aggregate(x_hbm, edge_hbm, zeros_hbm, out_hbm,
                   sidx, didx, rows, acc, gsem, ssem):
    c = lax.axis_index("c")
    s = lax.axis_index("s")
    wid = s * NC + c

    pltpu.sync_copy(zeros_hbm, acc.at[pl.ds(s * ROWS_PT, ROWS_PT)])

    def fire_gather(j, b):
        pltpu.async_copy(x_hbm.at[sidx.at[j]], rows.at[b], gsem)

    def fire_scatter(j, b):
        pltpu.async_copy(rows.at[b], acc.at[didx.at[j]], ssem, add=True)

    def drain(sem):
        pltpu.make_async_copy(rows.at[0], acc.at[didx.at[0]], sem).wait()

    def run_span(base, count):
        pltpu.sync_copy(edge_hbm.at[0, wid, pl.ds(base, count)],
                        sidx.at[pl.ds(0, count)])
        pltpu.sync_copy(edge_hbm.at[1, wid, pl.ds(base, count)],
                        didx.at[pl.ds(0, count)])

        fire_gather(0, 0)
        drain(gsem)
        fire_scatter(0, 0)
        fire_gather(1, 1)

        def body(j, carry):
            g = lax.rem(j, 2)
            drain(gsem)
            drain(ssem)
            fire_scatter(j, g)
            fire_gather(j + 1, 1 - g)
            return carry

        lax.fori_loop(1, count - 1, body, 0)

        gl = (count - 1) % 2
        drain(gsem)
        drain(ssem)
        fire_scatter(count - 1, gl)
        drain(ssem)

    plsc.subcore_barrier()
    run_span(0, HALF)
    run_span(HALF, NCHUNK - HALF)
    plsc.subcore_barrier()

    pltpu.sync_copy(
        acc.at[pl.ds(s * ROWS_PT, ROWS_PT)],
        out_hbm.at[c, pl.ds(s * ROWS_PT, ROWS_PT)],
    )


ROW_BLK = 400


def _mlp_body(x_ref, p_ref, w1_ref, b1_ref, w2_ref, b2_ref, o_ref):
    h = x_ref[...] + p_ref[0] + p_ref[1]
    h1 = jnp.dot(h, w1_ref[...], preferred_element_type=jnp.float32)
    h1 = jnp.maximum(h1 + b1_ref[...], 0.0)
    h2 = jnp.dot(h1, w2_ref[...], preferred_element_type=jnp.float32)
    h2 = jnp.maximum(h2 + b2_ref[...], 0.0)
    m = jnp.max(h2, axis=1, keepdims=True)
    e = h2 - m
    lse = jnp.log(jnp.sum(jnp.exp(e), axis=1, keepdims=True))
    o_ref[...] = e - lse


def kernel(x, edge_index, W1, b1, W2, b2):
    edges = edge_index.astype(jnp.int32).reshape(2, NW, NCHUNK, CHUNK)
    zeros = jnp.zeros((ROWS_PT, D_IN), jnp.float32)

    parts = _gin_aggregate(x, edges, zeros)

    grid = (N_NODES // ROW_BLK,)
    out = pl.pallas_call(
        _mlp_body,
        grid=grid,
        in_specs=[
            pl.BlockSpec((ROW_BLK, D_IN), lambda i: (i, 0)),
            pl.BlockSpec((NC, ROW_BLK, D_IN), lambda i: (0, i, 0)),
            pl.BlockSpec((D_IN, D_HID), lambda i: (0, 0)),
            pl.BlockSpec((1, D_HID), lambda i: (0, 0)),
            pl.BlockSpec((D_HID, D_OUT), lambda i: (0, 0)),
            pl.BlockSpec((1, D_OUT), lambda i: (0, 0)),
        ],
        out_specs=pl.BlockSpec((ROW_BLK, D_OUT), lambda i: (i, 0)),
        out_shape=jax.ShapeDtypeStruct((N_NODES, D_OUT), jnp.float32),
    )(x, parts, W1, b1.reshape(1, D_HID), W2, b2.reshape(1, D_OUT))
    return out


# speedup vs baseline: 1.3965x; 1.1725x over previous
import functools

import jax
import jax.numpy as jnp
from jax import lax
from jax.experimental import pallas as pl
from jax.experimental.pallas import tpu as pltpu
from jax.experimental.pallas import tpu_sc as plsc

N_NODES = 10000
N_EDGES = 320000
D_IN = 128
D_HID = 256
D_OUT = 64

NC = 2
NS = 16
NW = NC * NS
EPW = N_EDGES // NW
CHUNK = 80
NCHUNK = EPW // CHUNK
SPANS = (40, 40, 45)
SPAN_MAX = 48
NSLOT = 3
PAD_NODES = 10240
ROWS_PT = PAD_NODES // NS

_sc_mesh = plsc.VectorSubcoreMesh(
    core_axis_name="c", subcore_axis_name="s", num_cores=NC, num_subcores=NS
)


@functools.partial(
    pl.kernel,
    out_type=jax.ShapeDtypeStruct((NC, PAD_NODES, D_IN), jnp.float32),
    mesh=_sc_mesh,
    scratch_types=[
        pltpu.VMEM((SPAN_MAX, CHUNK), jnp.int32),
        pltpu.VMEM((SPAN_MAX, CHUNK), jnp.int32),
        pltpu.VMEM((NSLOT, CHUNK, D_IN), jnp.float32),
        pltpu.VMEM_SHARED((PAD_NODES, D_IN), jnp.float32),
        pltpu.SemaphoreType.DMA((NSLOT,)),
        pltpu.SemaphoreType.DMA((NSLOT,)),
    ],
)
def _gin_aggregate(x_hbm, edge_hbm, zeros_hbm, out_hbm,
                   sidx, didx, rows, acc, gsem, ssem):
    c = lax.axis_index("c")
    s = lax.axis_index("s")
    wid = s * NC + c

    pltpu.sync_copy(zeros_hbm, acc.at[pl.ds(s * ROWS_PT, ROWS_PT)])

    def fire_gather(j, b):
        pltpu.async_copy(x_hbm.at[sidx.at[j]], rows.at[b], gsem.at[b])

    def fire_scatter(j, b):
        pltpu.async_copy(rows.at[b], acc.at[didx.at[j]], ssem.at[b], add=True)

    def wait_g(b):
        pltpu.make_async_copy(rows.at[0], acc.at[didx.at[0]],
                              gsem.at[b]).wait()

    def wait_s(b):
        pltpu.make_async_copy(rows.at[0], acc.at[didx.at[0]],
                              ssem.at[b]).wait()

    def run_span(base, count):
        pltpu.sync_copy(edge_hbm.at[0, wid, pl.ds(base, count)],
                        sidx.at[pl.ds(0, count)])
        pltpu.sync_copy(edge_hbm.at[1, wid, pl.ds(base, count)],
                        didx.at[pl.ds(0, count)])

        def step(j, b):
            wait_s(b)
            fire_gather(j, b)
            wait_g((b + 2) % NSLOT)
            fire_scatter(j - 1, (b + 2) % NSLOT)

        fire_gather(0, 0)
        fire_gather(1, 1)
        wait_g(0)
        fire_scatter(0, 0)
        fire_gather(2, 2)
        wait_g(1)
        fire_scatter(1, 1)

        nmain = (count - NSLOT) // NSLOT
        ntail = (count - NSLOT) % NSLOT

        def blk(t, carry):
            j0 = NSLOT + t * NSLOT
            for b in range(NSLOT):
                step(j0 + b, b)
            return carry

        lax.fori_loop(0, nmain, blk, 0)

        for i in range(ntail):
            j = NSLOT + nmain * NSLOT + i
            step(j, j % NSLOT)

        wait_g((count - 1) % NSLOT)
        fire_scatter(count - 1, (count - 1) % NSLOT)
        for j in (count - 3, count - 2, count - 1):
            wait_s(j % NSLOT)

    plsc.subcore_barrier()
    off = 0
    for n in SPANS:
        run_span(off, n)
        off += n
    plsc.subcore_barrier()

    pltpu.sync_copy(
        acc.at[pl.ds(s * ROWS_PT, ROWS_PT)],
        out_hbm.at[c, pl.ds(s * ROWS_PT, ROWS_PT)],
    )


ROW_BLK = 400


def _mlp_body(x_ref, p_ref, w1_ref, b1_ref, w2_ref, b2_ref, o_ref):
    h = x_ref[...] + p_ref[0] + p_ref[1]
    h1 = jnp.dot(h, w1_ref[...], preferred_element_type=jnp.float32)
    h1 = jnp.maximum(h1 + b1_ref[...], 0.0)
    h2 = jnp.dot(h1, w2_ref[...], preferred_element_type=jnp.float32)
    h2 = jnp.maximum(h2 + b2_ref[...], 0.0)
    m = jnp.max(h2, axis=1, keepdims=True)
    e = h2 - m
    lse = jnp.log(jnp.sum(jnp.exp(e), axis=1, keepdims=True))
    o_ref[...] = e - lse


def kernel(x, edge_index, W1, b1, W2, b2):
    edges = edge_index.astype(jnp.int32).reshape(2, NW, NCHUNK, CHUNK)
    zeros = jnp.zeros((ROWS_PT, D_IN), jnp.float32)

    parts = _gin_aggregate(x, edges, zeros)

    grid = (N_NODES // ROW_BLK,)
    out = pl.pallas_call(
        _mlp_body,
        grid=grid,
        in_specs=[
            pl.BlockSpec((ROW_BLK, D_IN), lambda i: (i, 0)),
            pl.BlockSpec((NC, ROW_BLK, D_IN), lambda i: (0, i, 0)),
            pl.BlockSpec((D_IN, D_HID), lambda i: (0, 0)),
            pl.BlockSpec((1, D_HID), lambda i: (0, 0)),
            pl.BlockSpec((D_HID, D_OUT), lambda i: (0, 0)),
            pl.BlockSpec((1, D_OUT), lambda i: (0, 0)),
        ],
        out_specs=pl.BlockSpec((ROW_BLK, D_OUT), lambda i: (i, 0)),
        out_shape=jax.ShapeDtypeStruct((N_NODES, D_OUT), jnp.float32),
    )(x, parts, W1, b1.reshape(1, D_HID), W2, b2.reshape(1, D_OUT))
    return out


# speedup vs baseline: 1.5146x; 1.0846x over previous
import functools

import jax
import jax.numpy as jnp
from jax import lax
from jax.experimental import pallas as pl
from jax.experimental.pallas import tpu as pltpu
from jax.experimental.pallas import tpu_sc as plsc

N_NODES = 10000
N_EDGES = 320000
D_IN = 128
D_HID = 256
D_OUT = 64

NC = 2
NS = 16
NW = NC * NS
EPW = N_EDGES // NW
CHUNK = 80
NCHUNK = EPW // CHUNK
SPANS = (40, 40, 45)
SPAN_MAX = 48
NSLOT = 3
PAD_NODES = 10240
ROWS_PT = PAD_NODES // NS
LAST_X = N_NODES - (NS - 1) * ROWS_PT

_sc_mesh = plsc.VectorSubcoreMesh(
    core_axis_name="c", subcore_axis_name="s", num_cores=NC, num_subcores=NS
)


@functools.partial(
    pl.kernel,
    out_type=jax.ShapeDtypeStruct((NC, PAD_NODES, D_IN), jnp.float32),
    mesh=_sc_mesh,
    scratch_types=[
        pltpu.VMEM((SPAN_MAX, CHUNK), jnp.int32),
        pltpu.VMEM((SPAN_MAX, CHUNK), jnp.int32),
        pltpu.VMEM((NSLOT, CHUNK, D_IN), jnp.float32),
        pltpu.VMEM_SHARED((PAD_NODES, D_IN), jnp.float32),
        pltpu.SemaphoreType.DMA((NSLOT,)),
        pltpu.SemaphoreType.DMA((NSLOT,)),
    ],
)
def _gin_aggregate(x_hbm, edge_hbm, zeros_hbm, out_hbm,
                   sidx, didx, rows, acc, gsem, ssem):
    c = lax.axis_index("c")
    s = lax.axis_index("s")
    wid = s * NC + c

    @pl.when(c == 0)
    def _():
        @pl.when(s < NS - 1)
        def _():
            pltpu.sync_copy(x_hbm.at[pl.ds(s * ROWS_PT, ROWS_PT)],
                            acc.at[pl.ds(s * ROWS_PT, ROWS_PT)])

        @pl.when(s == NS - 1)
        def _():
            pltpu.sync_copy(x_hbm.at[pl.ds((NS - 1) * ROWS_PT, LAST_X)],
                            acc.at[pl.ds((NS - 1) * ROWS_PT, LAST_X)])
            pltpu.sync_copy(zeros_hbm.at[pl.ds(0, PAD_NODES - N_NODES)],
                            acc.at[pl.ds(N_NODES, PAD_NODES - N_NODES)])

    @pl.when(c != 0)
    def _():
        pltpu.sync_copy(zeros_hbm, acc.at[pl.ds(s * ROWS_PT, ROWS_PT)])

    def fire_gather(j, b):
        pltpu.async_copy(x_hbm.at[sidx.at[j]], rows.at[b], gsem.at[b])

    def fire_scatter(j, b):
        pltpu.async_copy(rows.at[b], acc.at[didx.at[j]], ssem.at[b], add=True)

    def wait_g(b):
        pltpu.make_async_copy(rows.at[0], acc.at[didx.at[0]],
                              gsem.at[b]).wait()

    def wait_s(b):
        pltpu.make_async_copy(rows.at[0], acc.at[didx.at[0]],
                              ssem.at[b]).wait()

    def run_span(base, count):
        pltpu.sync_copy(edge_hbm.at[0, wid, pl.ds(base, count)],
                        sidx.at[pl.ds(0, count)])
        pltpu.sync_copy(edge_hbm.at[1, wid, pl.ds(base, count)],
                        didx.at[pl.ds(0, count)])

        def step(j, b):
            wait_s(b)
            fire_gather(j, b)
            wait_g((b + 2) % NSLOT)
            fire_scatter(j - 1, (b + 2) % NSLOT)

        fire_gather(0, 0)
        fire_gather(1, 1)
        wait_g(0)
        fire_scatter(0, 0)
        fire_gather(2, 2)
        wait_g(1)
        fire_scatter(1, 1)

        nmain = (count - NSLOT) // NSLOT
        ntail = (count - NSLOT) % NSLOT

        def blk(t, carry):
            j0 = NSLOT + t * NSLOT
            for b in range(NSLOT):
                step(j0 + b, b)
            return carry

        lax.fori_loop(0, nmain, blk, 0)

        for i in range(ntail):
            j = NSLOT + nmain * NSLOT + i
            step(j, j % NSLOT)

        wait_g((count - 1) % NSLOT)
        fire_scatter(count - 1, (count - 1) % NSLOT)
        for j in (count - 3, count - 2, count - 1):
            wait_s(j % NSLOT)

    plsc.subcore_barrier()
    off = 0
    for n in SPANS:
        run_span(off, n)
        off += n
    plsc.subcore_barrier()

    pltpu.sync_copy(
        acc.at[pl.ds(s * ROWS_PT, ROWS_PT)],
        out_hbm.at[c, pl.ds(s * ROWS_PT, ROWS_PT)],
    )


ROW_BLK = 1000


def _mlp_body(p_ref, w1_ref, b1_ref, w2_ref, b2_ref, o_ref):
    h = p_ref[0] + p_ref[1]
    h1 = jnp.dot(h, w1_ref[...], preferred_element_type=jnp.float32)
    h1 = jnp.maximum(h1 + b1_ref[...], 0.0)
    h2 = jnp.dot(h1, w2_ref[...], preferred_element_type=jnp.float32)
    h2 = jnp.maximum(h2 + b2_ref[...], 0.0)
    m = jnp.max(h2, axis=1, keepdims=True)
    e = h2 - m
    lse = jnp.log(jnp.sum(jnp.exp(e), axis=1, keepdims=True))
    o_ref[...] = e - lse


def kernel(x, edge_index, W1, b1, W2, b2):
    edges = edge_index.astype(jnp.int32).reshape(2, NW, NCHUNK, CHUNK)
    zeros = jnp.zeros((ROWS_PT, D_IN), jnp.float32)

    parts = _gin_aggregate(x, edges, zeros)

    grid = (N_NODES // ROW_BLK,)
    out = pl.pallas_call(
        _mlp_body,
        grid=grid,
        in_specs=[
            pl.BlockSpec((NC, ROW_BLK, D_IN), lambda i: (0, i, 0)),
            pl.BlockSpec((D_IN, D_HID), lambda i: (0, 0)),
            pl.BlockSpec((1, D_HID), lambda i: (0, 0)),
            pl.BlockSpec((D_HID, D_OUT), lambda i: (0, 0)),
            pl.BlockSpec((1, D_OUT), lambda i: (0, 0)),
        ],
        out_specs=pl.BlockSpec((ROW_BLK, D_OUT), lambda i: (i, 0)),
        out_shape=jax.ShapeDtypeStruct((N_NODES, D_OUT), jnp.float32),
    )(parts, W1, b1.reshape(1, D_HID), W2, b2.reshape(1, D_OUT))
    return out
